# baseline (speedup 1.0000x reference)
import jax
import jax.numpy as jnp
from jax import lax
from jax.experimental import pallas as pl
from jax.experimental.pallas import tpu as pltpu
from jax.experimental.pallas import tpu_sc as plsc

VOCAB_SIZE = 32000
HIDDEN_SIZE = 4096
BATCH = 4
SEQ_LEN = 4096

NC = 2
NS = 16
NW = NC * NS

B = BATCH * SEQ_LEN
BPW = B // NW
K = 2
CH = BPW // K

_mesh = plsc.VectorSubcoreMesh(
    core_axis_name="c", subcore_axis_name="s", num_cores=NC, num_subcores=NS
)


NBUF = 8


@jax.jit
def _embed_gather(idx3, table):
    @pl.kernel(
        out_type=jax.ShapeDtypeStruct((B, HIDDEN_SIZE), jnp.float32),
        mesh=_mesh,
        scratch_types=[
            pltpu.VMEM((CH, K), jnp.int32),
            *[pltpu.VMEM((K, HIDDEN_SIZE), jnp.float32) for _ in range(NBUF)],
            *[pltpu.SemaphoreType.DMA for _ in range(2 * NBUF)],
        ],
    )
    def k(idx_hbm, table_hbm, out_hbm, idx_v, *scr):
        bufs = scr[:NBUF]
        gsems = scr[NBUF:2 * NBUF]
        wsems = scr[2 * NBUF:]
        wid = lax.axis_index("s") * NC + lax.axis_index("c")
        base = wid * BPW
        pltpu.sync_copy(idx_hbm.at[wid], idx_v)

        for b in range(NBUF):
            pltpu.async_copy(table_hbm.at[idx_v.at[b]], bufs[b], gsems[b])

        def step(g, _):
            for b in range(NBUF):
                j = g * NBUF + b
                pltpu.make_async_copy(
                    table_hbm.at[idx_v.at[j]], bufs[b], gsems[b]
                ).wait()
                pltpu.async_copy(
                    bufs[b], out_hbm.at[pl.ds(base + j * K, K)], wsems[b]
                )
            for b in range(NBUF):
                j = g * NBUF + b
                jn = j + NBUF
                pltpu.make_async_copy(
                    bufs[b], out_hbm.at[pl.ds(base + j * K, K)], wsems[b]
                ).wait()

                @pl.when(jn < CH)
                def _():
                    pltpu.async_copy(table_hbm.at[idx_v.at[jn]], bufs[b], gsems[b])

            return 0

        lax.fori_loop(0, CH // NBUF, step, 0)

    return k(idx3, table)


def kernel(input_ids, embed_weight):
    idx3 = input_ids.reshape(NW, CH, K)
    out = _embed_gather(idx3, embed_weight)
    return out.reshape(BATCH, SEQ_LEN, HIDDEN_SIZE)


# speedup vs baseline: 1.0079x; 1.0079x over previous
import jax
import jax.numpy as jnp
from jax import lax
from jax.experimental import pallas as pl
from jax.experimental.pallas import tpu as pltpu
from jax.experimental.pallas import tpu_sc as plsc

VOCAB_SIZE = 32000
HIDDEN_SIZE = 4096
BATCH = 4
SEQ_LEN = 4096

NC = 2
NS = 16
NW = NC * NS

B = BATCH * SEQ_LEN
BPW = B // NW

K = 4
NBUF = 4
SCM = 4

BR = 4
NBANK = 2
V = 4

M = 16
RS = M * SCM * K
RD = M * V * BR
CH_S = RS // K
NV = M * V
assert RS + RD == BPW

_mesh = plsc.VectorSubcoreMesh(
    core_axis_name="c", subcore_axis_name="s", num_cores=NC, num_subcores=NS
)


@jax.jit
def _embed_gather(idx_s3, idx_d2, table):
    @pl.kernel(
        out_type=jax.ShapeDtypeStruct((B, HIDDEN_SIZE), jnp.float32),
        mesh=_mesh,
        scratch_types=[
            pltpu.VMEM((CH_S, K), jnp.int32),
            pltpu.VMEM_SHARED((NS, RD), jnp.int32),
            pltpu.SMEM((RD,), jnp.int32),
            pltpu.VMEM_SHARED((NS, NBANK, BR, HIDDEN_SIZE), jnp.float32),
            *[pltpu.VMEM((K, HIDDEN_SIZE), jnp.float32) for _ in range(NBUF)],
            *[pltpu.SemaphoreType.DMA for _ in range(2 * NBUF)],
            *[pltpu.SemaphoreType.DMA for _ in range(2 * NBANK)],
        ],
    )
    def k(idx_s_hbm, idx_d_hbm, table_hbm, out_hbm, idx_v, spm_idx, smem_idx,
          spm, *scr):
        bufs = scr[:NBUF]
        gsems = scr[NBUF:2 * NBUF]
        wsems = scr[2 * NBUF:3 * NBUF]
        isems = scr[3 * NBUF:3 * NBUF + NBANK]
        osems = scr[3 * NBUF + NBANK:]
        sid = lax.axis_index("s")
        wid = sid * NC + lax.axis_index("c")
        base = wid * BPW

        pltpu.sync_copy(idx_s_hbm.at[wid], idx_v)
        pltpu.sync_copy(idx_d_hbm.at[wid], spm_idx.at[sid])
        pltpu.sync_copy(spm_idx.at[sid], smem_idx)

        def bank_wait(bk):
            pltpu.make_async_copy(
                table_hbm.at[pl.ds(0, BR)], spm.at[sid, bk], isems[bk]
            ).wait()

        def bank_out(bk, v_glob):
            pltpu.async_copy(
                spm.at[sid, bk],
                out_hbm.at[pl.ds(base + RS + v_glob * BR, BR)],
                osems[bk],
            )

        def bank_out_wait(bk, v_glob):
            pltpu.make_async_copy(
                spm.at[sid, bk],
                out_hbm.at[pl.ds(base + RS + v_glob * BR, BR)],
                osems[bk],
            ).wait()

        for b in range(NBUF):
            pltpu.async_copy(table_hbm.at[idx_v.at[b]], bufs[b], gsems[b])

        def macro(m, _):
            for i in range(SCM):
                c = m * SCM + i
                b = i
                pltpu.make_async_copy(
                    table_hbm.at[idx_v.at[c]], bufs[b], gsems[b]
                ).wait()
                pltpu.async_copy(
                    bufs[b], out_hbm.at[pl.ds(base + c * K, K)], wsems[b]
                )

            for v in range(V):
                bk = v % NBANK
                v_glob = m * V + v

                @pl.when(v_glob >= NBANK)
                def _drain():
                    bank_wait(bk)
                    bank_out(bk, v_glob - NBANK)
                    bank_out_wait(bk, v_glob - NBANK)

                for l in range(BR):
                    s = smem_idx[v_glob * BR + l]
                    pltpu.async_copy(
                        table_hbm.at[s], spm.at[sid, bk, l], isems[bk]
                    )

            for i in range(SCM):
                c = m * SCM + i
                b = i
                cn = c + NBUF
                pltpu.make_async_copy(
                    bufs[b], out_hbm.at[pl.ds(base + c * K, K)], wsems[b]
                ).wait()

                @pl.when(cn < CH_S)
                def _refill():
                    pltpu.async_copy(
                        table_hbm.at[idx_v.at[cn]], bufs[b], gsems[b]
                    )

            return 0

        lax.fori_loop(0, M, macro, 0)

        for bk in range(NBANK):
            v_glob = NV - NBANK + bk
            bank_wait(bk)
            bank_out(bk, v_glob)
            bank_out_wait(bk, v_glob)

    return k(idx_s3, idx_d2, table)


def kernel(input_ids, embed_weight):
    idx2 = input_ids.reshape(NW, BPW)
    idx_s3 = idx2[:, :RS].reshape(NW, CH_S, K)
    idx_d2 = idx2[:, RS:]
    out = _embed_gather(idx_s3, idx_d2, embed_weight)
    return out.reshape(BATCH, SEQ_LEN, HIDDEN_SIZE)
